# Initial kernel scaffold; baseline (speedup 1.0000x reference)
import jax
import jax.numpy as jnp
from jax.experimental import pallas as pl


def kernel(query_nodes, key_value_nodes, edge_index, proj_w, proj_b, attend_w, attend_b):
    raise NotImplementedError("write your pallas kernel here")


# speedup vs baseline: 6.5300x; 6.5300x over previous
import functools

import jax
import jax.numpy as jnp
from jax import lax
from jax.experimental import pallas as pl
from jax.experimental.pallas import tpu as pltpu
from jax.experimental.pallas import tpu_sc as plsc

NQ = 10000
NKV = 10000
E = 320000
D = 128

NC = 2
NS = 16
NW = NC * NS
L = 16
D_PER = D // NW
CHUNK = 2000


def _tc_project(q_ref, kvt_ref, w_ref, b_ref, aw_ref, ab_ref,
                sq_ref, skv_ref, kvpt_ref):
    W = w_ref[...]
    b = b_ref[...]
    aw = aw_ref[...]
    aw_q = aw[:D]
    aw_kv = aw[D:]
    kvpt = jnp.dot(W, kvt_ref[...], preferred_element_type=jnp.float32)
    kvpt = kvpt + b[:, None]
    kvpt_ref[...] = kvpt
    v_q = jnp.sum(aw_q[:, None] * W, axis=0)
    const = jnp.sum(b * aw_q) + ab_ref[0, 0]
    sq_ref[...] = jnp.sum(q_ref[...] * v_q[None, :], axis=1) + const
    skv_ref[...] = jnp.sum(kvpt * aw_kv[:, None], axis=0)


_project = pl.pallas_call(
    _tc_project,
    out_shape=[
        jax.ShapeDtypeStruct((NQ,), jnp.float32),
        jax.ShapeDtypeStruct((NKV,), jnp.float32),
        jax.ShapeDtypeStruct((D, NKV), jnp.float32),
    ],
)


_mesh = plsc.VectorSubcoreMesh(core_axis_name="c", subcore_axis_name="s")


@functools.partial(
    pl.kernel,
    out_type=jax.ShapeDtypeStruct((D, NQ), jnp.float32),
    mesh=_mesh,
    compiler_params=pltpu.CompilerParams(needs_layout_passes=False),
    scratch_types=[
        pltpu.VMEM((NQ,), jnp.float32),
        pltpu.VMEM((NKV,), jnp.float32),
        pltpu.VMEM((D_PER, NKV), jnp.float32),
        pltpu.VMEM((D_PER, NQ), jnp.float32),
        pltpu.VMEM((NQ,), jnp.float32),
        pltpu.VMEM((CHUNK,), jnp.int32),
        pltpu.VMEM((CHUNK,), jnp.int32),
    ],
)
def _sc_aggregate(qi_hbm, kvi_hbm, sq_hbm, skv_hbm, kvpt_hbm, acct_hbm,
                  sq_v, skv_v, kvp_v, acc_v, z_v, qib, kvib):
    wid = lax.axis_index("s") * NC + lax.axis_index("c")
    row0 = wid * D_PER

    pltpu.sync_copy(sq_hbm, sq_v)
    pltpu.sync_copy(skv_hbm, skv_v)
    pltpu.sync_copy(kvpt_hbm.at[pl.ds(row0, D_PER)], kvp_v)

    zeros = jnp.zeros((L,), jnp.float32)

    def zero_body(i, carry):
        sl = pl.ds(i * L, L)
        z_v[sl] = zeros
        for d in range(D_PER):
            acc_v[d, sl] = zeros
        return carry

    lax.fori_loop(0, NQ // L, zero_body, 0)

    def group_body(g, carry):
        sl = pl.ds(g * L, L)
        qi = qib[sl]
        kvi = kvib[sl]
        sq = plsc.load_gather(sq_v, [qi])
        skv = plsc.load_gather(skv_v, [kvi])
        e = sq + skv
        e = jnp.maximum(e, 0.2 * e)
        w = jnp.exp(e)
        plsc.addupdate_scatter(z_v, [qi], w)
        for d in range(D_PER):
            dv = jnp.full((L,), d, jnp.int32)
            col = plsc.load_gather(kvp_v, [dv, kvi])
            plsc.addupdate_scatter(acc_v, [dv, qi], w * col)
        return carry

    def chunk_body(c, carry):
        off = c * CHUNK
        pltpu.sync_copy(qi_hbm.at[pl.ds(off, CHUNK)], qib)
        pltpu.sync_copy(kvi_hbm.at[pl.ds(off, CHUNK)], kvib)
        lax.fori_loop(0, CHUNK // L, group_body, 0)
        return carry

    lax.fori_loop(0, E // CHUNK, chunk_body, 0)

    def scale_body(i, carry):
        sl = pl.ds(i * L, L)
        r = 1.0 / (z_v[sl] + 1e-10)
        for d in range(D_PER):
            acc_v[d, sl] = acc_v[d, sl] * r
        return carry

    lax.fori_loop(0, NQ // L, scale_body, 0)

    pltpu.sync_copy(acc_v, acct_hbm.at[pl.ds(row0, D_PER)])


def kernel(query_nodes, key_value_nodes, edge_index, proj_w, proj_b,
           attend_w, attend_b):
    kvt = key_value_nodes.T
    ab = jnp.reshape(attend_b, (1, 1))
    sq, skv, kvpt = _project(query_nodes, kvt, proj_w, proj_b, attend_w, ab)
    acct = _sc_aggregate(edge_index[0], edge_index[1], sq, skv, kvpt)
    return acct.T


# speedup vs baseline: 11.0533x; 1.6927x over previous
import functools

import jax
import jax.numpy as jnp
from jax import lax
from jax.experimental import pallas as pl
from jax.experimental.pallas import tpu as pltpu
from jax.experimental.pallas import tpu_sc as plsc

NQ = 10000
NKV = 10000
E = 320000
D = 128

NC = 2
NS = 16
NW = NC * NS
L = 16
D_PER = D // NW
CHUNK = 2000


def _tc_project(q_ref, kvt_ref, w_ref, b_ref, aw_ref, ab_ref,
                sq_ref, skv_ref, kvpt_ref):
    W = w_ref[...]
    b = b_ref[...]
    aw = aw_ref[...]
    aw_q = aw[:D]
    aw_kv = aw[D:]
    kvpt = jnp.dot(W, kvt_ref[...], preferred_element_type=jnp.float32)
    kvpt = kvpt + b[:, None]
    kvpt_ref[...] = kvpt
    v_q = jnp.sum(aw_q[:, None] * W, axis=0)
    const = jnp.sum(b * aw_q) + ab_ref[0, 0]
    sq_ref[...] = jnp.sum(q_ref[...] * v_q[None, :], axis=1) + const
    skv_ref[...] = jnp.sum(kvpt * aw_kv[:, None], axis=0)


_project = pl.pallas_call(
    _tc_project,
    out_shape=[
        jax.ShapeDtypeStruct((NQ,), jnp.float32),
        jax.ShapeDtypeStruct((NKV,), jnp.float32),
        jax.ShapeDtypeStruct((D, NKV), jnp.float32),
    ],
)


_mesh = plsc.VectorSubcoreMesh(core_axis_name="c", subcore_axis_name="s")


@functools.partial(
    pl.kernel,
    out_type=jax.ShapeDtypeStruct((D, NQ), jnp.float32),
    mesh=_mesh,
    compiler_params=pltpu.CompilerParams(needs_layout_passes=False),
    scratch_types=[
        pltpu.VMEM((NQ,), jnp.float32),
        pltpu.VMEM((NKV,), jnp.float32),
        pltpu.VMEM((D_PER, NKV), jnp.float32),
        pltpu.VMEM((D_PER, NQ), jnp.float32),
        pltpu.VMEM((NQ,), jnp.float32),
        pltpu.VMEM((CHUNK,), jnp.int32),
        pltpu.VMEM((CHUNK,), jnp.int32),
    ],
)
def _sc_aggregate(qi_hbm, kvi_hbm, sq_hbm, skv_hbm, kvpt_hbm, acct_hbm,
                  sq_v, skv_v, kvp_v, acc_v, z_v, qib, kvib):
    wid = lax.axis_index("s") * NC + lax.axis_index("c")
    row0 = wid * D_PER

    pltpu.sync_copy(sq_hbm, sq_v)
    pltpu.sync_copy(skv_hbm, skv_v)
    pltpu.sync_copy(kvpt_hbm.at[pl.ds(row0, D_PER)], kvp_v)

    zeros = jnp.zeros((L,), jnp.float32)

    def zero_body(i, carry):
        sl = pl.ds(i * L, L)
        z_v[sl] = zeros
        for d in range(D_PER):
            acc_v[d, sl] = zeros
        return carry

    lax.fori_loop(0, NQ // L, zero_body, 0)

    def chunk_body(c, carry):
        off = c * CHUNK
        pltpu.sync_copy(qi_hbm.at[pl.ds(off, CHUNK)], qib)
        pltpu.sync_copy(kvi_hbm.at[pl.ds(off, CHUNK)], kvib)

        @plsc.parallel_loop(0, CHUNK // L, unroll=8)
        def group_body(g):
            sl = pl.ds(g * L, L)
            qi = qib[sl]
            kvi = kvib[sl]
            sq = plsc.load_gather(sq_v, [qi])
            skv = plsc.load_gather(skv_v, [kvi])
            e = sq + skv
            e = jnp.maximum(e, 0.2 * e)
            w = jnp.exp(e)
            plsc.addupdate_scatter(z_v, [qi], w)
            for d in range(D_PER):
                dv = jnp.full((L,), d, jnp.int32)
                col = plsc.load_gather(kvp_v, [dv, kvi])
                plsc.addupdate_scatter(acc_v, [dv, qi], w * col)

        return carry

    lax.fori_loop(0, E // CHUNK, chunk_body, 0)

    def scale_body(i, carry):
        sl = pl.ds(i * L, L)
        r = 1.0 / (z_v[sl] + 1e-10)
        for d in range(D_PER):
            acc_v[d, sl] = acc_v[d, sl] * r
        return carry

    lax.fori_loop(0, NQ // L, scale_body, 0)

    pltpu.sync_copy(acc_v, acct_hbm.at[pl.ds(row0, D_PER)])


def kernel(query_nodes, key_value_nodes, edge_index, proj_w, proj_b,
           attend_w, attend_b):
    kvt = key_value_nodes.T
    ab = jnp.reshape(attend_b, (1, 1))
    sq, skv, kvpt = _project(query_nodes, kvt, proj_w, proj_b, attend_w, ab)
    acct = _sc_aggregate(edge_index[0], edge_index[1], sq, skv, kvpt)
    return acct.T
